# baseline (speedup 1.0000x reference)
import jax
import jax.numpy as jnp
from jax.experimental import pallas as pl
from jax.experimental.pallas import tpu as pltpu

_CHUNK = 1024
_NBUF = 8
_LAG = 4


def _body(x_hbm, o_hbm, bufs, in_sems, out_sems):
    nchunk = x_hbm.shape[0] // _CHUNK

    def load(i, b):
        return pltpu.make_async_copy(
            x_hbm.at[pl.ds(i * _CHUNK, _CHUNK)], bufs.at[b], in_sems.at[b])

    def store(i, b):
        return pltpu.make_async_copy(
            bufs.at[b], o_hbm.at[pl.ds(i * _CHUNK, _CHUNK)], out_sems.at[b])

    for b in range(min(_NBUF, nchunk)):
        load(b, b).start()
    for i in range(nchunk):
        b = i % _NBUF
        load(i, b).wait()
        store(i, b).start()
        j = i - _LAG
        if 0 <= j and j + _NBUF < nchunk:
            bj = j % _NBUF
            store(j, bj).wait()
            load(j + _NBUF, bj).start()
    for j in range(max(0, nchunk - _NBUF), nchunk):
        store(j, j % _NBUF).wait()


def kernel(x):
    b, s, d = x.shape
    x2 = x.reshape(b * s, d)
    out = pl.pallas_call(
        _body,
        out_shape=jax.ShapeDtypeStruct(x2.shape, x2.dtype),
        in_specs=[pl.BlockSpec(memory_space=pl.ANY)],
        out_specs=pl.BlockSpec(memory_space=pl.ANY),
        scratch_shapes=[
            pltpu.VMEM((_NBUF, _CHUNK, d), x.dtype),
            pltpu.SemaphoreType.DMA((_NBUF,)),
            pltpu.SemaphoreType.DMA((_NBUF,)),
        ],
        compiler_params=pltpu.CompilerParams(
            vmem_limit_bytes=100 * 1024 * 1024,
        ),
    )(x2)
    return out.reshape(b, s, d)
